# baseline (speedup 1.0000x reference)
import functools

import jax
import jax.numpy as jnp
from jax import lax
from jax.experimental import pallas as pl
from jax.experimental.pallas import tpu as pltpu
from jax.experimental.pallas import tpu_sc as plsc

NN = 100000
EE = 1600000
HID = 64
NW = 32
NP = 102400
EPW = EE // NW
GCH = 2000
EROWS = EE // 128
SROWS_PW = 392
EROWS_P = SROWS_PW * NW
EP = EROWS_P * 128
E2 = 2 * EE
CL = 12800
NEB = EE // CL
NSB = E2 // CL
GCH2 = 4000

_f32 = jnp.float32
_i32 = jnp.int32

def _c32(v):
    return jnp.int32(v)


_MESH = plsc.VectorSubcoreMesh(core_axis_name="c", subcore_axis_name="s")
_SC_PARAMS = pltpu.CompilerParams(needs_layout_passes=False)


def _sc_gather_body(nx_hbm, row_hbm, col_hbm, outr_hbm, outc_hbm,
                    nx_v, rbuf, cbuf, orbuf, ocbuf):
    cid = lax.axis_index("c")
    sid = lax.axis_index("s")
    wid = sid * 2 + cid
    base = wid * _c32(EPW)
    pltpu.sync_copy(nx_hbm, nx_v)

    def chunk(ci, _):
        off = base + ci * _c32(GCH)
        pltpu.sync_copy(row_hbm.at[pl.ds(off, GCH)], rbuf)
        pltpu.sync_copy(col_hbm.at[pl.ds(off, GCH)], cbuf)

        def vec(vi, _):
            o = vi * _c32(16)
            idx_r = rbuf[pl.ds(o, 16)]
            orbuf[pl.ds(o, 16)] = plsc.load_gather(nx_v, [idx_r])
            idx_c = cbuf[pl.ds(o, 16)]
            ocbuf[pl.ds(o, 16)] = plsc.load_gather(nx_v, [idx_c])
            return _c32(0)

        lax.fori_loop(_c32(0), _c32(GCH // 16), vec, _c32(0))
        pltpu.sync_copy(orbuf, outr_hbm.at[pl.ds(off, GCH)])
        pltpu.sync_copy(ocbuf, outc_hbm.at[pl.ds(off, GCH)])
        return _c32(0)

    lax.fori_loop(_c32(0), _c32(EPW // GCH), chunk, _c32(0))


_sc_gather = pl.kernel(
    _sc_gather_body,
    out_type=(jax.ShapeDtypeStruct((EE,), _f32),
              jax.ShapeDtypeStruct((EE,), _f32)),
    mesh=_MESH,
    compiler_params=_SC_PARAMS,
    scratch_types=[
        pltpu.VMEM((NP,), _f32),
        pltpu.VMEM((GCH,), _i32),
        pltpu.VMEM((GCH,), _i32),
        pltpu.VMEM((GCH,), _f32),
        pltpu.VMEM((GCH,), _f32),
    ],
)


def _sc_segsum_body(idx_hbm, val_hbm, zeros_hbm, out_hbm,
                    idxbuf, valbuf, acc):
    cid = lax.axis_index("c")
    sid = lax.axis_index("s")
    wid = sid * 2 + cid

    @pl.when(sid == 0)
    def _():
        pltpu.sync_copy(zeros_hbm, acc)

    plsc.subcore_barrier()

    rbase = wid * _c32(SROWS_PW)
    pltpu.sync_copy(idx_hbm.at[pl.ds(rbase, SROWS_PW)], idxbuf)
    pltpu.sync_copy(val_hbm.at[pl.ds(rbase, SROWS_PW)], valbuf)

    def rowfn(j, _):
        pltpu.sync_copy(valbuf.at[j], acc.at[idxbuf.at[j]], add=True)
        return _c32(0)

    lax.fori_loop(_c32(0), _c32(SROWS_PW), rowfn, _c32(0))
    plsc.subcore_barrier()

    @pl.when(sid == 0)
    def _():
        pltpu.sync_copy(acc, out_hbm.at[cid])


_sc_segsum = pl.kernel(
    _sc_segsum_body,
    out_type=jax.ShapeDtypeStruct((2, NP), _f32),
    mesh=_MESH,
    compiler_params=_SC_PARAMS,
    scratch_types=[
        pltpu.VMEM((SROWS_PW, 128), _i32),
        pltpu.VMEM((SROWS_PW, 128), _f32),
        pltpu.VMEM_SHARED((NP,), _f32),
    ],
)


def _sc_perm_gather_body(t0_hbm, t1_hbm, t2_hbm, idx_hbm,
                         o0_hbm, o1_hbm, o2_hbm,
                         idxbuf, b0, b1, b2, sem):
    cid = lax.axis_index("c")
    sid = lax.axis_index("s")
    wid = sid * 2 + cid
    base = wid * _c32(E2 // NW)

    def chunk(ci, _):
        off = base + ci * _c32(GCH2)
        pltpu.sync_copy(idx_hbm.at[pl.ds(off, GCH2)], idxbuf)
        pltpu.async_copy(t0_hbm.at[idxbuf], b0, sem).wait()
        pltpu.async_copy(t1_hbm.at[idxbuf], b1, sem).wait()
        pltpu.async_copy(t2_hbm.at[idxbuf], b2, sem).wait()
        pltpu.sync_copy(b0, o0_hbm.at[pl.ds(off, GCH2)])
        pltpu.sync_copy(b1, o1_hbm.at[pl.ds(off, GCH2)])
        pltpu.sync_copy(b2, o2_hbm.at[pl.ds(off, GCH2)])
        return _c32(0)

    lax.fori_loop(_c32(0), _c32((E2 // NW) // GCH2), chunk, _c32(0))


_sc_perm_gather = pl.kernel(
    _sc_perm_gather_body,
    out_type=(jax.ShapeDtypeStruct((E2,), _f32),
              jax.ShapeDtypeStruct((E2,), _i32),
              jax.ShapeDtypeStruct((E2,), _i32)),
    mesh=_MESH,
    compiler_params=_SC_PARAMS,
    scratch_types=[
        pltpu.VMEM((GCH2,), _i32),
        pltpu.VMEM((GCH2,), _f32),
        pltpu.VMEM((GCH2,), _i32),
        pltpu.VMEM((GCH2,), _i32),
        pltpu.SemaphoreType.DMA,
    ],
)


def _tc_mlp_body(nfeat, *refs):
    feats = refs[:nfeat]
    w1t, b1, w2, b2, out = refs[nfeat:nfeat + 5]
    h = b1[...]
    for j in range(nfeat):
        h = h + w1t[:, j:j + 1] * feats[j][0]
    h = jnp.maximum(h, 0.0)
    out[0] = jnp.sum(h * w2[...], axis=0, keepdims=True) + b2[...]


def _edge_mlp(feats, eW1, eb1, eW2, eb2):
    nfeat = len(feats)
    nblk = feats[0].shape[0] // CL
    fspec = pl.BlockSpec((1, 1, CL), lambda i: (i, _c32(0), _c32(0)))
    wspec = lambda shp: pl.BlockSpec(shp, lambda i: (_c32(0), _c32(0)))
    return pl.pallas_call(
        functools.partial(_tc_mlp_body, nfeat),
        grid=(nblk,),
        in_specs=[fspec] * nfeat + [
            wspec((HID, nfeat)), wspec((HID, 1)), wspec((HID, 1)), wspec((1, 1)),
        ],
        out_specs=fspec,
        out_shape=jax.ShapeDtypeStruct((nblk, 1, CL), _f32),
    )(*[f.reshape(nblk, 1, CL) for f in feats],
      eW1.T, eb1[:, None], eW2, eb2[None, :]).reshape(-1)


def _tc_node_body(nx, s0, s1, cnt, w1t, b1, w2, b2, out):
    agg = (s0[0] + s1[0]) / jnp.maximum(cnt[0], 1.0)
    h = b1[...] + w1t[:, 0:1] * nx[0] + w1t[:, 1:2] * agg
    h = jnp.maximum(h, 0.0)
    out[0] = jnp.sum(h * w2[...], axis=0, keepdims=True) + b2[...]


def _node_mlp(nx, s2, cnt, nW1, nb1, nW2, nb2):
    nblk = NP // CL
    fspec = pl.BlockSpec((1, 1, CL), lambda i: (i, _c32(0), _c32(0)))
    wspec = lambda shp: pl.BlockSpec(shp, lambda i: (_c32(0), _c32(0)))
    return pl.pallas_call(
        _tc_node_body,
        grid=(nblk,),
        in_specs=[fspec, fspec, fspec, fspec,
                  wspec((HID, 2)), wspec((HID, 1)), wspec((HID, 1)), wspec((1, 1))],
        out_specs=fspec,
        out_shape=jax.ShapeDtypeStruct((nblk, 1, CL), _f32),
    )(nx.reshape(nblk, 1, CL), s2[0].reshape(nblk, 1, CL), s2[1].reshape(nblk, 1, CL),
      cnt.reshape(nblk, 1, CL), nW1.T, nb1[:, None], nW2, nb2[None, :]).reshape(-1)


def _tc_sym_body(row, col, ee, kf, ks, vals, fo, so):
    pid = pl.program_id(0)
    islo = pid < NEB
    r = row[0]
    c = col[0]
    v = ee[0]
    ev = jnp.where(r == c, jnp.sqrt(jnp.exp(v)), v)
    first = jnp.where(islo, r, c)
    second = jnp.where(islo, c, r)
    m = first <= second
    kf[0] = jnp.where(m, first, NN)
    ks[0] = jnp.where(m, second, 0)
    vals[0] = jnp.where(m, ev, 0.0)
    fo[0] = first
    so[0] = second


def _sym_prepare(row32, col32, ee):
    espec = pl.BlockSpec((1, 1, CL), lambda i: (i % _c32(NEB), _c32(0), _c32(0)))
    ospec = pl.BlockSpec((1, 1, CL), lambda i: (i, _c32(0), _c32(0)))
    shp_i = jax.ShapeDtypeStruct((NSB, 1, CL), _i32)
    shp_f = jax.ShapeDtypeStruct((NSB, 1, CL), _f32)
    outs = pl.pallas_call(
        _tc_sym_body,
        grid=(NSB,),
        in_specs=[espec, espec, espec],
        out_specs=[ospec] * 5,
        out_shape=[shp_i, shp_i, shp_f, shp_i, shp_i],
    )(row32.reshape(NEB, 1, CL), col32.reshape(NEB, 1, CL), ee.reshape(NEB, 1, CL))
    return [o.reshape(-1) for o in outs]


def _tc_isstart_body(kf, ks, kfp, ksp, out):
    out[0] = ((kf[0] != kfp[0]) | (ks[0] != ksp[0])).astype(_i32)


def _is_start(skf, sks, skfp, sksp):
    spec = pl.BlockSpec((1, 1, CL), lambda i: (i, _c32(0), _c32(0)))
    return pl.pallas_call(
        _tc_isstart_body,
        grid=(NSB,),
        in_specs=[spec] * 4,
        out_specs=spec,
        out_shape=jax.ShapeDtypeStruct((NSB, 1, CL), _i32),
    )(skf.reshape(NSB, 1, CL), sks.reshape(NSB, 1, CL),
      skfp.reshape(NSB, 1, CL), sksp.reshape(NSB, 1, CL)).reshape(-1)


def _tc_coalesce_body(sv, iss, issn, out, carry):
    pid = pl.program_id(0)

    @pl.when(pid == 0)
    def _():
        carry[0, 0] = 0.0

    cin = carry[0, 0]
    t = sv[0]
    g = 1.0 - issn[0].astype(_f32)
    zero = jnp.zeros((1, CL), _f32)
    lane = lax.broadcasted_iota(_i32, (1, CL), 1)
    pad_t = jnp.where(lane == 0, cin, 0.0)
    tw = jnp.concatenate([t, pad_t], axis=1)
    gw = jnp.concatenate([g, zero], axis=1)
    d = 1
    while d < CL + 1:
        tws = jnp.concatenate([tw[:, d:], jnp.zeros((1, d), _f32)], axis=1)
        gws = jnp.concatenate([gw[:, d:], jnp.zeros((1, d), _f32)], axis=1)
        tw = tw + gw * tws
        gw = gw * gws
        d *= 2
    tt = tw[:, :CL]
    st = iss[0].astype(_f32)
    out[0] = st * tt
    carry[0, 0] = jnp.sum(jnp.where(lane == 0, (1.0 - st) * tt, 0.0))


def _coalesce(svals, iss, issn):
    spec = pl.BlockSpec((1, 1, CL), lambda i: (_c32(NSB - 1) - i, _c32(0), _c32(0)))
    return pl.pallas_call(
        _tc_coalesce_body,
        grid=(NSB,),
        in_specs=[spec] * 3,
        out_specs=spec,
        out_shape=jax.ShapeDtypeStruct((NSB, 1, CL), _f32),
        scratch_shapes=[pltpu.SMEM((1, 1), _f32)],
    )(svals.reshape(NSB, 1, CL), iss.reshape(NSB, 1, CL),
      issn.reshape(NSB, 1, CL)).reshape(-1)


from jax.experimental.compute_on import compute_on


@compute_on("tpu_sparsecore")
@jax.jit
def _sc_sort3(k, p1, p2):
    return lax.sort((k, p1, p2), num_keys=1, is_stable=True)


def kernel(x, edge_index, edge_attr, params):
    row32 = edge_index[0].astype(_i32)
    col32 = edge_index[1].astype(_i32)
    ea = edge_attr[:, 0].astype(_f32)

    row_pad2d = jnp.pad(row32, (0, EP - EE)).reshape(EROWS_P, 128)
    ones_pad2d = jnp.pad(jnp.ones((EE,), _f32), (0, EP - EE)).reshape(EROWS_P, 128)
    zeros_np = jnp.zeros((NP,), _f32)

    cnt2 = _sc_segsum(row_pad2d, ones_pad2d, zeros_np)
    cnt = cnt2[0] + cnt2[1]

    nx = jnp.pad(x[:, 0].astype(_f32), (0, NP - NN))
    ee = ea
    for li, p in enumerate(params):
        eW1, eb1, eW2, eb2, nW1, nb1, nW2, nb2 = p
        xr, xc = _sc_gather(nx, row32, col32)
        if li == 0:
            feats = [xr, xc, ea]
        else:
            feats = [xr, xc, ee, ea]
        ee = _edge_mlp(feats, eW1, eb1, eW2, eb2)
        ee_pad2d = jnp.pad(ee, (0, EP - EE)).reshape(EROWS_P, 128)
        s2 = _sc_segsum(row_pad2d, ee_pad2d, zeros_np)
        nx = _node_mlp(nx, s2, cnt, nW1, nb1, nW2, nb2)

    kf, ks, vals, fo, so = _sym_prepare(row32, col32, ee)
    tpos = lax.iota(_i32, E2)
    sks1, skf1, st1 = _sc_sort3(ks, kf, tpos)
    skf, sks, st = _sc_sort3(skf1, sks1, st1)

    svals, sfo, sso = _sc_perm_gather(vals, fo, so, st)

    skfp = jnp.concatenate([jnp.full((1,), -1, _i32), skf[:-1]])
    sksp = jnp.concatenate([jnp.full((1,), -1, _i32), sks[:-1]])
    iss = _is_start(skf, sks, skfp, sksp)
    issn = jnp.concatenate([iss[1:], jnp.ones((1,), _i32)])
    out_vals = _coalesce(svals, iss, issn)

    sidx = jnp.stack([sfo.astype(jnp.int64), sso.astype(jnp.int64)])
    return sidx, out_vals


# speedup vs baseline: 1.5862x; 1.5862x over previous
import functools

import jax
import jax.numpy as jnp
from jax import lax
from jax.experimental import pallas as pl
from jax.experimental.pallas import tpu as pltpu
from jax.experimental.pallas import tpu_sc as plsc

NN = 100000
EE = 1600000
HID = 64
NW = 32
NP = 102400
EPW = EE // NW
GCH = 2000
EROWS = EE // 128
SROWS_PW = 392
EROWS_P = SROWS_PW * NW
EP = EROWS_P * 128
E2 = 2 * EE
CL = 12800
NEB = EE // CL
NSB = E2 // CL
GCH2 = 4000

_f32 = jnp.float32
_i32 = jnp.int32

def _c32(v):
    return jnp.int32(v)


_MESH = plsc.VectorSubcoreMesh(core_axis_name="c", subcore_axis_name="s")
_SC_PARAMS = pltpu.CompilerParams(needs_layout_passes=False)


def _sc_gather_body(nx_hbm, row_hbm, col_hbm, outr_hbm, outc_hbm,
                    nx_v, rbuf, cbuf, orbuf, ocbuf):
    cid = lax.axis_index("c")
    sid = lax.axis_index("s")
    wid = sid * 2 + cid
    base = wid * _c32(EPW)
    pltpu.sync_copy(nx_hbm, nx_v)

    def chunk(ci, _):
        off = base + ci * _c32(GCH)
        pltpu.sync_copy(row_hbm.at[pl.ds(off, GCH)], rbuf)
        pltpu.sync_copy(col_hbm.at[pl.ds(off, GCH)], cbuf)

        def vec(vi, _):
            o = vi * _c32(16)
            idx_r = rbuf[pl.ds(o, 16)]
            orbuf[pl.ds(o, 16)] = plsc.load_gather(nx_v, [idx_r])
            idx_c = cbuf[pl.ds(o, 16)]
            ocbuf[pl.ds(o, 16)] = plsc.load_gather(nx_v, [idx_c])
            return _c32(0)

        lax.fori_loop(_c32(0), _c32(GCH // 16), vec, _c32(0))
        pltpu.sync_copy(orbuf, outr_hbm.at[pl.ds(off, GCH)])
        pltpu.sync_copy(ocbuf, outc_hbm.at[pl.ds(off, GCH)])
        return _c32(0)

    lax.fori_loop(_c32(0), _c32(EPW // GCH), chunk, _c32(0))


_sc_gather = pl.kernel(
    _sc_gather_body,
    out_type=(jax.ShapeDtypeStruct((EE,), _f32),
              jax.ShapeDtypeStruct((EE,), _f32)),
    mesh=_MESH,
    compiler_params=_SC_PARAMS,
    scratch_types=[
        pltpu.VMEM((NP,), _f32),
        pltpu.VMEM((GCH,), _i32),
        pltpu.VMEM((GCH,), _i32),
        pltpu.VMEM((GCH,), _f32),
        pltpu.VMEM((GCH,), _f32),
    ],
)


def _sc_segsum_body(idx_hbm, val_hbm, zeros_hbm, out_hbm,
                    idxbuf, valbuf, acc):
    cid = lax.axis_index("c")
    sid = lax.axis_index("s")
    wid = sid * 2 + cid

    @pl.when(sid == 0)
    def _():
        pltpu.sync_copy(zeros_hbm, acc)

    plsc.subcore_barrier()

    rbase = wid * _c32(SROWS_PW)
    pltpu.sync_copy(idx_hbm.at[pl.ds(rbase, SROWS_PW)], idxbuf)
    pltpu.sync_copy(val_hbm.at[pl.ds(rbase, SROWS_PW)], valbuf)

    def rowfn(j, _):
        pltpu.sync_copy(valbuf.at[j], acc.at[idxbuf.at[j]], add=True)
        return _c32(0)

    lax.fori_loop(_c32(0), _c32(SROWS_PW), rowfn, _c32(0))
    plsc.subcore_barrier()

    @pl.when(sid == 0)
    def _():
        pltpu.sync_copy(acc, out_hbm.at[cid])


_sc_segsum = pl.kernel(
    _sc_segsum_body,
    out_type=jax.ShapeDtypeStruct((2, NP), _f32),
    mesh=_MESH,
    compiler_params=_SC_PARAMS,
    scratch_types=[
        pltpu.VMEM((SROWS_PW, 128), _i32),
        pltpu.VMEM((SROWS_PW, 128), _f32),
        pltpu.VMEM_SHARED((NP,), _f32),
    ],
)


def _sc_perm_gather_body(t0_hbm, t1_hbm, t2_hbm, idx_hbm,
                         o0_hbm, o1_hbm, o2_hbm,
                         idxbuf, b0, b1, b2, sem):
    cid = lax.axis_index("c")
    sid = lax.axis_index("s")
    wid = sid * 2 + cid
    base = wid * _c32(E2 // NW)

    def chunk(ci, _):
        off = base + ci * _c32(GCH2)
        pltpu.sync_copy(idx_hbm.at[pl.ds(off, GCH2)], idxbuf)
        pltpu.async_copy(t0_hbm.at[idxbuf], b0, sem).wait()
        pltpu.async_copy(t1_hbm.at[idxbuf], b1, sem).wait()
        pltpu.async_copy(t2_hbm.at[idxbuf], b2, sem).wait()
        pltpu.sync_copy(b0, o0_hbm.at[pl.ds(off, GCH2)])
        pltpu.sync_copy(b1, o1_hbm.at[pl.ds(off, GCH2)])
        pltpu.sync_copy(b2, o2_hbm.at[pl.ds(off, GCH2)])
        return _c32(0)

    lax.fori_loop(_c32(0), _c32((E2 // NW) // GCH2), chunk, _c32(0))


_sc_perm_gather = pl.kernel(
    _sc_perm_gather_body,
    out_type=(jax.ShapeDtypeStruct((E2,), _f32),
              jax.ShapeDtypeStruct((E2,), _i32),
              jax.ShapeDtypeStruct((E2,), _i32)),
    mesh=_MESH,
    compiler_params=_SC_PARAMS,
    scratch_types=[
        pltpu.VMEM((GCH2,), _i32),
        pltpu.VMEM((GCH2,), _f32),
        pltpu.VMEM((GCH2,), _i32),
        pltpu.VMEM((GCH2,), _i32),
        pltpu.SemaphoreType.DMA,
    ],
)


def _tc_mlp_body(nfeat, *refs):
    feats = refs[:nfeat]
    w1t, b1, w2, b2, out = refs[nfeat:nfeat + 5]
    h = b1[...]
    for j in range(nfeat):
        h = h + w1t[:, j:j + 1] * feats[j][0]
    h = jnp.maximum(h, 0.0)
    out[0] = jnp.sum(h * w2[...], axis=0, keepdims=True) + b2[...]


def _edge_mlp(feats, eW1, eb1, eW2, eb2):
    nfeat = len(feats)
    nblk = feats[0].shape[0] // CL
    fspec = pl.BlockSpec((1, 1, CL), lambda i: (i, _c32(0), _c32(0)))
    wspec = lambda shp: pl.BlockSpec(shp, lambda i: (_c32(0), _c32(0)))
    return pl.pallas_call(
        functools.partial(_tc_mlp_body, nfeat),
        grid=(nblk,),
        in_specs=[fspec] * nfeat + [
            wspec((HID, nfeat)), wspec((HID, 1)), wspec((HID, 1)), wspec((1, 1)),
        ],
        out_specs=fspec,
        out_shape=jax.ShapeDtypeStruct((nblk, 1, CL), _f32),
    )(*[f.reshape(nblk, 1, CL) for f in feats],
      eW1.T, eb1[:, None], eW2, eb2[None, :]).reshape(-1)


def _tc_node_body(nx, s0, s1, cnt, w1t, b1, w2, b2, out):
    agg = (s0[0] + s1[0]) / jnp.maximum(cnt[0], 1.0)
    h = b1[...] + w1t[:, 0:1] * nx[0] + w1t[:, 1:2] * agg
    h = jnp.maximum(h, 0.0)
    out[0] = jnp.sum(h * w2[...], axis=0, keepdims=True) + b2[...]


def _node_mlp(nx, s2, cnt, nW1, nb1, nW2, nb2):
    nblk = NP // CL
    fspec = pl.BlockSpec((1, 1, CL), lambda i: (i, _c32(0), _c32(0)))
    wspec = lambda shp: pl.BlockSpec(shp, lambda i: (_c32(0), _c32(0)))
    return pl.pallas_call(
        _tc_node_body,
        grid=(nblk,),
        in_specs=[fspec, fspec, fspec, fspec,
                  wspec((HID, 2)), wspec((HID, 1)), wspec((HID, 1)), wspec((1, 1))],
        out_specs=fspec,
        out_shape=jax.ShapeDtypeStruct((nblk, 1, CL), _f32),
    )(nx.reshape(nblk, 1, CL), s2[0].reshape(nblk, 1, CL), s2[1].reshape(nblk, 1, CL),
      cnt.reshape(nblk, 1, CL), nW1.T, nb1[:, None], nW2, nb2[None, :]).reshape(-1)


def _tc_sym_body(row, col, ee, kf, ks, vals, fo, so):
    pid = pl.program_id(0)
    islo = pid < NEB
    r = row[0]
    c = col[0]
    v = ee[0]
    ev = jnp.where(r == c, jnp.sqrt(jnp.exp(v)), v)
    first = jnp.where(islo, r, c)
    second = jnp.where(islo, c, r)
    m = first <= second
    kf[0] = jnp.where(m, first, NN)
    ks[0] = jnp.where(m, second, 0)
    vals[0] = jnp.where(m, ev, 0.0)
    fo[0] = first
    so[0] = second


def _sym_prepare(row32, col32, ee):
    espec = pl.BlockSpec((1, 1, CL), lambda i: (i % _c32(NEB), _c32(0), _c32(0)))
    ospec = pl.BlockSpec((1, 1, CL), lambda i: (i, _c32(0), _c32(0)))
    shp_i = jax.ShapeDtypeStruct((NSB, 1, CL), _i32)
    shp_f = jax.ShapeDtypeStruct((NSB, 1, CL), _f32)
    outs = pl.pallas_call(
        _tc_sym_body,
        grid=(NSB,),
        in_specs=[espec, espec, espec],
        out_specs=[ospec] * 5,
        out_shape=[shp_i, shp_i, shp_f, shp_i, shp_i],
    )(row32.reshape(NEB, 1, CL), col32.reshape(NEB, 1, CL), ee.reshape(NEB, 1, CL))
    return [o.reshape(-1) for o in outs]


def _tc_isstart_body(kf, ks, kfp, ksp, out):
    out[0] = ((kf[0] != kfp[0]) | (ks[0] != ksp[0])).astype(_i32)


def _is_start(skf, sks, skfp, sksp):
    spec = pl.BlockSpec((1, 1, CL), lambda i: (i, _c32(0), _c32(0)))
    return pl.pallas_call(
        _tc_isstart_body,
        grid=(NSB,),
        in_specs=[spec] * 4,
        out_specs=spec,
        out_shape=jax.ShapeDtypeStruct((NSB, 1, CL), _i32),
    )(skf.reshape(NSB, 1, CL), sks.reshape(NSB, 1, CL),
      skfp.reshape(NSB, 1, CL), sksp.reshape(NSB, 1, CL)).reshape(-1)


def _tc_coalesce_body(sv, iss, issn, out, carry):
    pid = pl.program_id(0)

    @pl.when(pid == 0)
    def _():
        carry[0, 0] = 0.0

    cin = carry[0, 0]
    t = sv[0]
    g = 1.0 - issn[0].astype(_f32)
    zero = jnp.zeros((1, CL), _f32)
    lane = lax.broadcasted_iota(_i32, (1, CL), 1)
    pad_t = jnp.where(lane == 0, cin, 0.0)
    tw = jnp.concatenate([t, pad_t], axis=1)
    gw = jnp.concatenate([g, zero], axis=1)
    d = 1
    while d < CL + 1:
        tws = jnp.concatenate([tw[:, d:], jnp.zeros((1, d), _f32)], axis=1)
        gws = jnp.concatenate([gw[:, d:], jnp.zeros((1, d), _f32)], axis=1)
        tw = tw + gw * tws
        gw = gw * gws
        d *= 2
    tt = tw[:, :CL]
    st = iss[0].astype(_f32)
    out[0] = st * tt
    carry[0, 0] = jnp.sum(jnp.where(lane == 0, (1.0 - st) * tt, 0.0))


def _coalesce(svals, iss, issn):
    spec = pl.BlockSpec((1, 1, CL), lambda i: (_c32(NSB - 1) - i, _c32(0), _c32(0)))
    return pl.pallas_call(
        _tc_coalesce_body,
        grid=(NSB,),
        in_specs=[spec] * 3,
        out_specs=spec,
        out_shape=jax.ShapeDtypeStruct((NSB, 1, CL), _f32),
        scratch_shapes=[pltpu.SMEM((1, 1), _f32)],
    )(svals.reshape(NSB, 1, CL), iss.reshape(NSB, 1, CL),
      issn.reshape(NSB, 1, CL)).reshape(-1)


def kernel(x, edge_index, edge_attr, params):
    row32 = edge_index[0].astype(_i32)
    col32 = edge_index[1].astype(_i32)
    ea = edge_attr[:, 0].astype(_f32)

    row_pad2d = jnp.pad(row32, (0, EP - EE)).reshape(EROWS_P, 128)
    ones_pad2d = jnp.pad(jnp.ones((EE,), _f32), (0, EP - EE)).reshape(EROWS_P, 128)
    zeros_np = jnp.zeros((NP,), _f32)

    cnt2 = _sc_segsum(row_pad2d, ones_pad2d, zeros_np)
    cnt = cnt2[0] + cnt2[1]

    nx = jnp.pad(x[:, 0].astype(_f32), (0, NP - NN))
    ee = ea
    for li, p in enumerate(params):
        eW1, eb1, eW2, eb2, nW1, nb1, nW2, nb2 = p
        xr, xc = _sc_gather(nx, row32, col32)
        if li == 0:
            feats = [xr, xc, ea]
        else:
            feats = [xr, xc, ee, ea]
        ee = _edge_mlp(feats, eW1, eb1, eW2, eb2)
        ee_pad2d = jnp.pad(ee, (0, EP - EE)).reshape(EROWS_P, 128)
        s2 = _sc_segsum(row_pad2d, ee_pad2d, zeros_np)
        nx = _node_mlp(nx, s2, cnt, nW1, nb1, nW2, nb2)

    kf, ks, vals, fo, so = _sym_prepare(row32, col32, ee)
    tpos = lax.iota(_i32, E2)
    skf, sks, st = lax.sort((kf, ks, tpos), num_keys=2, is_stable=True)

    svals, sfo, sso = _sc_perm_gather(vals, fo, so, st)

    skfp = jnp.concatenate([jnp.full((1,), -1, _i32), skf[:-1]])
    sksp = jnp.concatenate([jnp.full((1,), -1, _i32), sks[:-1]])
    iss = _is_start(skf, sks, skfp, sksp)
    issn = jnp.concatenate([iss[1:], jnp.ones((1,), _i32)])
    out_vals = _coalesce(svals, iss, issn)

    sidx = jnp.stack([sfo.astype(jnp.int64), sso.astype(jnp.int64)])
    return sidx, out_vals
